# baseline (speedup 1.0000x reference)
import functools

import jax
import jax.numpy as jnp
from jax import lax
from jax.experimental import pallas as pl
from jax.experimental.pallas import tpu as pltpu
from jax.experimental.pallas import tpu_sc as plsc

N = 10000
E = 160000
D_HID = 128

N_PAD = 10112
E_PAD = 163840
CHUNKS = 40
CHUNK = 128
ROWS_PER_TILE = N_PAD // 16
N_ZERO_ROWS = N_PAD - N


BM = 1264


def _mm_body(x_ref, w_ref, o_ref):
    o_ref[...] = lax.dot_general(
        x_ref[...], w_ref[...], (((1,), (1,)), ((), ())),
        preferred_element_type=jnp.float32)


def _matmul(xp, W):
    m, k = xp.shape
    n = W.shape[0]
    return pl.pallas_call(
        _mm_body,
        grid=(m // BM,),
        in_specs=[pl.BlockSpec((BM, k), lambda i: (i, 0)),
                  pl.BlockSpec((n, k), lambda i: (0, 0))],
        out_specs=pl.BlockSpec((BM, n), lambda i: (i, 0)),
        out_shape=jax.ShapeDtypeStruct((m, n), jnp.float32),
    )(xp, W)


def _mm_relu_body(p_ref, w_ref, o_ref):
    a = jnp.maximum(p_ref[0] + p_ref[1], 0.0)
    o_ref[...] = lax.dot_general(
        a, w_ref[...], (((1,), (1,)), ((), ())),
        preferred_element_type=jnp.float32)


def _matmul_relu(p, W):
    _, m, k = p.shape
    n = W.shape[0]
    return pl.pallas_call(
        _mm_relu_body,
        grid=(m // BM,),
        in_specs=[pl.BlockSpec((2, BM, k), lambda i: (0, i, 0)),
                  pl.BlockSpec((n, k), lambda i: (0, 0))],
        out_specs=pl.BlockSpec((BM, n), lambda i: (i, 0)),
        out_shape=jax.ShapeDtypeStruct((m, n), jnp.float32),
    )(p, W)


def _sum2_body(p_ref, o_ref):
    o_ref[...] = p_ref[0] + p_ref[1]


def _sum2(p):
    _, m, n = p.shape
    return pl.pallas_call(
        _sum2_body,
        grid=(m // BM,),
        in_specs=[pl.BlockSpec((2, BM, n), lambda i: (0, i, 0))],
        out_specs=pl.BlockSpec((BM, n), lambda i: (i, 0)),
        out_shape=jax.ShapeDtypeStruct((m, n), jnp.float32),
    )(p)


_MESH = plsc.VectorSubcoreMesh(core_axis_name="c", subcore_axis_name="s")

NBUF = 2


@functools.partial(
    pl.kernel,
    mesh=_MESH,
    out_type=jax.ShapeDtypeStruct((2, N_PAD, D_HID), jnp.float32),
    scratch_types=[
        pltpu.VMEM((CHUNKS, CHUNK), jnp.int32),
        pltpu.VMEM((CHUNKS, CHUNK), jnp.int32),
        pltpu.VMEM((NBUF, CHUNK, D_HID), jnp.float32),
        pltpu.VMEM((8, D_HID), jnp.float32),
        pltpu.VMEM_SHARED((N_PAD, D_HID), jnp.float32),
        pltpu.SemaphoreType.DMA,
        pltpu.SemaphoreType.DMA,
        pltpu.SemaphoreType.DMA,
        pltpu.SemaphoreType.DMA,
        pltpu.SemaphoreType.DMA,
        pltpu.SemaphoreType.DMA,
    ],
)
def _sc_aggregate(h_hbm, src_hbm, dst_hbm, out_hbm,
                  src_v, dst_v, rows_v, zbuf, acc,
                  g0a, g0b, g1a, g1b, zsem, isem):
    gsems = [[g0a, g0b], [g1a, g1b]]
    HALF = CHUNK // 2
    c = lax.axis_index("c")
    s = lax.axis_index("s")
    wid = s * 2 + c
    row0 = s * ROWS_PER_TILE

    icp0 = pltpu.async_copy(src_hbm.at[wid], src_v, isem)
    icp1 = pltpu.async_copy(dst_hbm.at[wid], dst_v, isem)

    zeros = jnp.zeros((16,), jnp.float32)
    for i in range(8):
        for j in range(D_HID // 16):
            zbuf[i, pl.ds(j * 16, 16)] = zeros

    def zfire_body(i, carry):
        pltpu.async_copy(zbuf, acc.at[pl.ds(row0 + i * 8, 8)], zsem)
        return carry

    lax.fori_loop(0, ROWS_PER_TILE // 8, zfire_body, 0)
    pltpu.make_async_copy(h_hbm.at[pl.ds(0, ROWS_PER_TILE)],
                          acc.at[pl.ds(row0, ROWS_PER_TILE)], zsem).wait()

    icp0.wait()
    icp1.wait()

    def gissue(j, b):
        for hh in range(2):
            pltpu.async_copy(
                h_hbm.at[src_v.at[j, pl.ds(hh * HALF, HALF)]],
                rows_v.at[b, pl.ds(hh * HALF, HALF)], gsems[b][hh])

    def gwait(b):
        for hh in range(2):
            pltpu.make_async_copy(
                h_hbm.at[pl.ds(0, HALF)],
                rows_v.at[b, pl.ds(hh * HALF, HALF)], gsems[b][hh]).wait()

    for b in range(NBUF):
        gissue(b, b)
    plsc.subcore_barrier()

    def edge_body(jj, carry):
        for b in range(NBUF):
            j = jj * NBUF + b
            gwait(b)
            pltpu.sync_copy(rows_v.at[b], acc.at[dst_v.at[j]], add=True)
            gissue(j + NBUF, b)
        return carry

    lax.fori_loop(0, CHUNKS // NBUF - 1, edge_body, 0)
    for b in range(NBUF):
        j = CHUNKS - NBUF + b
        gwait(b)
        pltpu.sync_copy(rows_v.at[b], acc.at[dst_v.at[j]], add=True)
    plsc.subcore_barrier()

    pltpu.sync_copy(acc.at[pl.ds(row0, ROWS_PER_TILE)],
                    out_hbm.at[c, pl.ds(row0, ROWS_PER_TILE)])


def kernel(x, edge_index, W1, W2, W3):
    src = edge_index[0].astype(jnp.int32)
    dst = edge_index[1].astype(jnp.int32)
    pad_idx = N + (jnp.arange(E_PAD - E, dtype=jnp.int32) % N_ZERO_ROWS)
    srcp = jnp.concatenate([src, pad_idx]).reshape(32, CHUNKS, CHUNK)
    dstp = jnp.concatenate([dst, pad_idx]).reshape(32, CHUNKS, CHUNK)
    xp = jnp.pad(x, ((0, N_PAD - N), (0, 0)))

    h = _matmul(xp, W1)
    p = _sc_aggregate(h, srcp, dstp)
    h = _matmul_relu(p, W2)
    p = _sc_aggregate(h, srcp, dstp)
    h = _matmul_relu(p, W3)
    p = _sc_aggregate(h, srcp, dstp)
    out = _sum2(p)
    return out[:N]


# speedup vs baseline: 1.0176x; 1.0176x over previous
import functools

import jax
import jax.numpy as jnp
from jax import lax
from jax.experimental import pallas as pl
from jax.experimental.pallas import tpu as pltpu
from jax.experimental.pallas import tpu_sc as plsc

N = 10000
E = 160000
D_HID = 128

N_PAD = 10112
E_PAD = 163840
CHUNKS = 40
CHUNK = 128
ROWS_PER_TILE = N_PAD // 16
N_ZERO_ROWS = N_PAD - N


BM = 1264


def _mm_body(x_ref, w_ref, o_ref):
    o_ref[...] = lax.dot_general(
        x_ref[...], w_ref[...], (((1,), (1,)), ((), ())),
        preferred_element_type=jnp.float32)


def _matmul(xp, W):
    m, k = xp.shape
    n = W.shape[0]
    return pl.pallas_call(
        _mm_body,
        grid=(m // BM,),
        in_specs=[pl.BlockSpec((BM, k), lambda i: (i, 0)),
                  pl.BlockSpec((n, k), lambda i: (0, 0))],
        out_specs=pl.BlockSpec((BM, n), lambda i: (i, 0)),
        out_shape=jax.ShapeDtypeStruct((m, n), jnp.float32),
    )(xp, W)


def _mm_relu_body(p_ref, w_ref, o_ref):
    a = jnp.maximum(p_ref[0] + p_ref[1], 0.0)
    o_ref[...] = lax.dot_general(
        a, w_ref[...], (((1,), (1,)), ((), ())),
        preferred_element_type=jnp.float32)


def _matmul_relu(p, W):
    _, m, k = p.shape
    n = W.shape[0]
    return pl.pallas_call(
        _mm_relu_body,
        grid=(m // BM,),
        in_specs=[pl.BlockSpec((2, BM, k), lambda i: (0, i, 0)),
                  pl.BlockSpec((n, k), lambda i: (0, 0))],
        out_specs=pl.BlockSpec((BM, n), lambda i: (i, 0)),
        out_shape=jax.ShapeDtypeStruct((m, n), jnp.float32),
    )(p, W)


def _sum2_body(p_ref, o_ref):
    o_ref[...] = p_ref[0] + p_ref[1]


def _sum2(p):
    _, m, n = p.shape
    return pl.pallas_call(
        _sum2_body,
        grid=(m // BM,),
        in_specs=[pl.BlockSpec((2, BM, n), lambda i: (0, i, 0))],
        out_specs=pl.BlockSpec((BM, n), lambda i: (i, 0)),
        out_shape=jax.ShapeDtypeStruct((m, n), jnp.float32),
    )(p)


_MESH = plsc.VectorSubcoreMesh(core_axis_name="c", subcore_axis_name="s")

NBUF = 2


@functools.partial(
    pl.kernel,
    mesh=_MESH,
    out_type=jax.ShapeDtypeStruct((2, N_PAD, D_HID), jnp.float32),
    scratch_types=[
        pltpu.VMEM((CHUNKS, CHUNK), jnp.int32),
        pltpu.VMEM((CHUNKS, CHUNK), jnp.int32),
        pltpu.VMEM((NBUF, CHUNK, D_HID), jnp.float32),
        pltpu.VMEM((8, D_HID), jnp.float32),
        pltpu.VMEM_SHARED((N_PAD, D_HID), jnp.float32),
        pltpu.SemaphoreType.DMA,
        pltpu.SemaphoreType.DMA,
        pltpu.SemaphoreType.DMA,
        pltpu.SemaphoreType.DMA,
    ],
)
def _sc_aggregate(h_hbm, src_hbm, dst_hbm, out_hbm,
                  src_v, dst_v, rows_v, zbuf, acc, g0, g1, zsem, isem):
    gsems = [g0, g1]
    c = lax.axis_index("c")
    s = lax.axis_index("s")
    wid = s * 2 + c
    row0 = s * ROWS_PER_TILE

    icp0 = pltpu.async_copy(src_hbm.at[wid], src_v, isem)
    icp1 = pltpu.async_copy(dst_hbm.at[wid], dst_v, isem)

    zeros = jnp.zeros((16,), jnp.float32)
    for i in range(8):
        for j in range(D_HID // 16):
            zbuf[i, pl.ds(j * 16, 16)] = zeros

    def zfire_body(i, carry):
        pltpu.async_copy(zbuf, acc.at[pl.ds(row0 + i * 8, 8)], zsem)
        return carry

    lax.fori_loop(0, ROWS_PER_TILE // 8, zfire_body, 0)
    pltpu.make_async_copy(h_hbm.at[pl.ds(0, ROWS_PER_TILE)],
                          acc.at[pl.ds(row0, ROWS_PER_TILE)], zsem).wait()

    icp0.wait()
    icp1.wait()
    for b in range(NBUF):
        pltpu.async_copy(h_hbm.at[src_v.at[b]], rows_v.at[b], gsems[b])
    plsc.subcore_barrier()

    def edge_body(jj, carry):
        for b in range(NBUF):
            j = jj * NBUF + b
            pltpu.make_async_copy(
                h_hbm.at[pl.ds(0, CHUNK)], rows_v.at[b], gsems[b]).wait()
            pltpu.sync_copy(rows_v.at[b], acc.at[dst_v.at[j]], add=True)
            pltpu.async_copy(
                h_hbm.at[src_v.at[j + NBUF]], rows_v.at[b], gsems[b])
        return carry

    lax.fori_loop(0, CHUNKS // NBUF - 1, edge_body, 0)
    for b in range(NBUF):
        j = CHUNKS - NBUF + b
        pltpu.make_async_copy(
            h_hbm.at[pl.ds(0, CHUNK)], rows_v.at[b], gsems[b]).wait()
        pltpu.sync_copy(rows_v.at[b], acc.at[dst_v.at[j]], add=True)
    plsc.subcore_barrier()

    pltpu.sync_copy(acc.at[pl.ds(row0, ROWS_PER_TILE)],
                    out_hbm.at[c, pl.ds(row0, ROWS_PER_TILE)])


def kernel(x, edge_index, W1, W2, W3):
    src = edge_index[0].astype(jnp.int32)
    dst = edge_index[1].astype(jnp.int32)
    pad_idx = N + (jnp.arange(E_PAD - E, dtype=jnp.int32) % N_ZERO_ROWS)
    srcp = jnp.concatenate([src, pad_idx]).reshape(32, CHUNKS, CHUNK)
    dstp = jnp.concatenate([dst, pad_idx]).reshape(32, CHUNKS, CHUNK)
    xp = jnp.pad(x, ((0, N_PAD - N), (0, 0)))

    h = _matmul(xp, W1)
    p = _sc_aggregate(h, srcp, dstp)
    h = _matmul_relu(p, W2)
    p = _sc_aggregate(h, srcp, dstp)
    h = _matmul_relu(p, W3)
    p = _sc_aggregate(h, srcp, dstp)
    out = _sum2(p)
    return out[:N]


# speedup vs baseline: 1.0590x; 1.0407x over previous
import functools

import jax
import jax.numpy as jnp
from jax import lax
from jax.experimental import pallas as pl
from jax.experimental.pallas import tpu as pltpu
from jax.experimental.pallas import tpu_sc as plsc

N = 10000
E = 160000
D_HID = 128

N_PAD = 10112
E_PAD = 163840
CHUNKS = 40
CHUNK = 128
ROWS_PER_TILE = N_PAD // 16
N_ZERO_ROWS = N_PAD - N


BM = 10112


def _mm_body(x_ref, w_ref, o_ref):
    o_ref[...] = lax.dot_general(
        x_ref[...], w_ref[...], (((1,), (1,)), ((), ())),
        preferred_element_type=jnp.float32)


def _matmul(xp, W):
    m, k = xp.shape
    n = W.shape[0]
    return pl.pallas_call(
        _mm_body,
        grid=(m // BM,),
        in_specs=[pl.BlockSpec((BM, k), lambda i: (i, 0)),
                  pl.BlockSpec((n, k), lambda i: (0, 0))],
        out_specs=pl.BlockSpec((BM, n), lambda i: (i, 0)),
        out_shape=jax.ShapeDtypeStruct((m, n), jnp.float32),
    )(xp, W)


def _mm_relu_body(p_ref, w_ref, o_ref):
    a = jnp.maximum(p_ref[0] + p_ref[1], 0.0)
    o_ref[...] = lax.dot_general(
        a, w_ref[...], (((1,), (1,)), ((), ())),
        preferred_element_type=jnp.float32)


def _matmul_relu(p, W):
    _, m, k = p.shape
    n = W.shape[0]
    return pl.pallas_call(
        _mm_relu_body,
        grid=(m // BM,),
        in_specs=[pl.BlockSpec((2, BM, k), lambda i: (0, i, 0)),
                  pl.BlockSpec((n, k), lambda i: (0, 0))],
        out_specs=pl.BlockSpec((BM, n), lambda i: (i, 0)),
        out_shape=jax.ShapeDtypeStruct((m, n), jnp.float32),
    )(p, W)


def _sum2_body(p_ref, o_ref):
    o_ref[...] = p_ref[0] + p_ref[1]


def _sum2(p):
    _, m, n = p.shape
    return pl.pallas_call(
        _sum2_body,
        grid=(m // BM,),
        in_specs=[pl.BlockSpec((2, BM, n), lambda i: (0, i, 0))],
        out_specs=pl.BlockSpec((BM, n), lambda i: (i, 0)),
        out_shape=jax.ShapeDtypeStruct((m, n), jnp.float32),
    )(p)


_MESH = plsc.VectorSubcoreMesh(core_axis_name="c", subcore_axis_name="s")

NBUF = 2


@functools.partial(
    pl.kernel,
    mesh=_MESH,
    out_type=jax.ShapeDtypeStruct((2, N_PAD, D_HID), jnp.float32),
    scratch_types=[
        pltpu.VMEM((CHUNKS, CHUNK), jnp.int32),
        pltpu.VMEM((CHUNKS, CHUNK), jnp.int32),
        pltpu.VMEM((NBUF, CHUNK, D_HID), jnp.float32),
        pltpu.VMEM((8, D_HID), jnp.float32),
        pltpu.VMEM_SHARED((N_PAD, D_HID), jnp.float32),
        pltpu.SemaphoreType.DMA,
        pltpu.SemaphoreType.DMA,
        pltpu.SemaphoreType.DMA,
        pltpu.SemaphoreType.DMA,
    ],
)
def _sc_aggregate(h_hbm, src_hbm, dst_hbm, out_hbm,
                  src_v, dst_v, rows_v, zbuf, acc, g0, g1, zsem, isem):
    gsems = [g0, g1]
    c = lax.axis_index("c")
    s = lax.axis_index("s")
    wid = s * 2 + c
    row0 = s * ROWS_PER_TILE

    icp0 = pltpu.async_copy(src_hbm.at[wid], src_v, isem)
    icp1 = pltpu.async_copy(dst_hbm.at[wid], dst_v, isem)

    zeros = jnp.zeros((16,), jnp.float32)
    for i in range(8):
        for j in range(D_HID // 16):
            zbuf[i, pl.ds(j * 16, 16)] = zeros

    def zfire_body(i, carry):
        pltpu.async_copy(zbuf, acc.at[pl.ds(row0 + i * 8, 8)], zsem)
        return carry

    lax.fori_loop(0, ROWS_PER_TILE // 8, zfire_body, 0)
    pltpu.make_async_copy(h_hbm.at[pl.ds(0, ROWS_PER_TILE)],
                          acc.at[pl.ds(row0, ROWS_PER_TILE)], zsem).wait()

    icp0.wait()
    icp1.wait()
    for b in range(NBUF):
        pltpu.async_copy(h_hbm.at[src_v.at[b]], rows_v.at[b], gsems[b])
    plsc.subcore_barrier()

    def edge_body(jj, carry):
        for b in range(NBUF):
            j = jj * NBUF + b
            pltpu.make_async_copy(
                h_hbm.at[pl.ds(0, CHUNK)], rows_v.at[b], gsems[b]).wait()
            pltpu.sync_copy(rows_v.at[b], acc.at[dst_v.at[j]], add=True)
            pltpu.async_copy(
                h_hbm.at[src_v.at[j + NBUF]], rows_v.at[b], gsems[b])
        return carry

    lax.fori_loop(0, CHUNKS // NBUF - 1, edge_body, 0)
    for b in range(NBUF):
        j = CHUNKS - NBUF + b
        pltpu.make_async_copy(
            h_hbm.at[pl.ds(0, CHUNK)], rows_v.at[b], gsems[b]).wait()
        pltpu.sync_copy(rows_v.at[b], acc.at[dst_v.at[j]], add=True)
    plsc.subcore_barrier()

    pltpu.sync_copy(acc.at[pl.ds(row0, ROWS_PER_TILE)],
                    out_hbm.at[c, pl.ds(row0, ROWS_PER_TILE)])


def kernel(x, edge_index, W1, W2, W3):
    src = edge_index[0].astype(jnp.int32)
    dst = edge_index[1].astype(jnp.int32)
    pad_idx = N + (jnp.arange(E_PAD - E, dtype=jnp.int32) % N_ZERO_ROWS)
    srcp = jnp.concatenate([src, pad_idx]).reshape(32, CHUNKS, CHUNK)
    dstp = jnp.concatenate([dst, pad_idx]).reshape(32, CHUNKS, CHUNK)
    xp = jnp.pad(x, ((0, N_PAD - N), (0, 0)))

    h = _matmul(xp, W1)
    p = _sc_aggregate(h, srcp, dstp)
    h = _matmul_relu(p, W2)
    p = _sc_aggregate(h, srcp, dstp)
    h = _matmul_relu(p, W3)
    p = _sc_aggregate(h, srcp, dstp)
    out = _sum2(p)
    return out[:N]


# speedup vs baseline: 1.1172x; 1.0550x over previous
import functools

import jax
import jax.numpy as jnp
from jax import lax
from jax.experimental import pallas as pl
from jax.experimental.pallas import tpu as pltpu
from jax.experimental.pallas import tpu_sc as plsc

N = 10000
E = 160000
D_HID = 128

N_PAD = 10112
E_PAD = 163840
CHUNKS = 40
CHUNK = 128
ROWS_PER_TILE = N_PAD // 16
N_ZERO_ROWS = N_PAD - N


BM = 10112


def _mm_body(x_ref, w_ref, o_ref):
    o_ref[pl.ds(0, N), :] = lax.dot_general(
        x_ref[...], w_ref[...], (((1,), (1,)), ((), ())),
        preferred_element_type=jnp.float32)
    o_ref[pl.ds(N, N_PAD - N), :] = jnp.zeros(
        (N_PAD - N, D_HID), jnp.float32)


def _matmul(x, W):
    m, k = x.shape
    n = W.shape[0]
    return pl.pallas_call(
        _mm_body,
        grid=(1,),
        in_specs=[pl.BlockSpec((m, k), lambda i: (0, 0)),
                  pl.BlockSpec((n, k), lambda i: (0, 0))],
        out_specs=pl.BlockSpec((N_PAD, n), lambda i: (0, 0)),
        out_shape=jax.ShapeDtypeStruct((N_PAD, n), jnp.float32),
    )(x, W)


def _mm_relu_body(p_ref, w_ref, o_ref):
    a = jnp.maximum(p_ref[0] + p_ref[1], 0.0)
    o_ref[...] = lax.dot_general(
        a, w_ref[...], (((1,), (1,)), ((), ())),
        preferred_element_type=jnp.float32)


def _matmul_relu(p, W):
    _, m, k = p.shape
    n = W.shape[0]
    return pl.pallas_call(
        _mm_relu_body,
        grid=(m // BM,),
        in_specs=[pl.BlockSpec((2, BM, k), lambda i: (0, i, 0)),
                  pl.BlockSpec((n, k), lambda i: (0, 0))],
        out_specs=pl.BlockSpec((BM, n), lambda i: (i, 0)),
        out_shape=jax.ShapeDtypeStruct((m, n), jnp.float32),
    )(p, W)


def _sum2_body(p_ref, o_ref):
    o_ref[...] = p_ref[0, pl.ds(0, N), :] + p_ref[1, pl.ds(0, N), :]


def _sum2(p):
    _, m, n = p.shape
    return pl.pallas_call(
        _sum2_body,
        grid=(1,),
        in_specs=[pl.BlockSpec((2, m, n), lambda i: (0, 0, 0))],
        out_specs=pl.BlockSpec((N, n), lambda i: (0, 0)),
        out_shape=jax.ShapeDtypeStruct((N, n), jnp.float32),
    )(p)


_MESH = plsc.VectorSubcoreMesh(core_axis_name="c", subcore_axis_name="s")

NBUF = 2


@functools.partial(
    pl.kernel,
    mesh=_MESH,
    out_type=jax.ShapeDtypeStruct((2, N_PAD, D_HID), jnp.float32),
    scratch_types=[
        pltpu.VMEM((CHUNKS, CHUNK), jnp.int32),
        pltpu.VMEM((CHUNKS, CHUNK), jnp.int32),
        pltpu.VMEM((NBUF, CHUNK, D_HID), jnp.float32),
        pltpu.VMEM((8, D_HID), jnp.float32),
        pltpu.VMEM_SHARED((N_PAD, D_HID), jnp.float32),
        pltpu.SemaphoreType.DMA,
        pltpu.SemaphoreType.DMA,
        pltpu.SemaphoreType.DMA,
        pltpu.SemaphoreType.DMA,
    ],
)
def _sc_aggregate(h_hbm, src_hbm, dst_hbm, out_hbm,
                  src_v, dst_v, rows_v, zbuf, acc, g0, g1, zsem, isem):
    gsems = [g0, g1]
    c = lax.axis_index("c")
    s = lax.axis_index("s")
    wid = s * 2 + c
    row0 = s * ROWS_PER_TILE

    icp0 = pltpu.async_copy(src_hbm.at[wid], src_v, isem)
    icp1 = pltpu.async_copy(dst_hbm.at[wid], dst_v, isem)

    zeros = jnp.zeros((16,), jnp.float32)
    for i in range(8):
        for j in range(D_HID // 16):
            zbuf[i, pl.ds(j * 16, 16)] = zeros

    def zfire_body(i, carry):
        pltpu.async_copy(zbuf, acc.at[pl.ds(row0 + i * 8, 8)], zsem)
        return carry

    lax.fori_loop(0, ROWS_PER_TILE // 8, zfire_body, 0)
    pltpu.make_async_copy(h_hbm.at[pl.ds(0, ROWS_PER_TILE)],
                          acc.at[pl.ds(row0, ROWS_PER_TILE)], zsem).wait()

    icp0.wait()
    icp1.wait()
    for b in range(NBUF):
        pltpu.async_copy(h_hbm.at[src_v.at[b]], rows_v.at[b], gsems[b])
    plsc.subcore_barrier()

    def edge_body(jj, carry):
        for b in range(NBUF):
            j = jj * NBUF + b
            pltpu.make_async_copy(
                h_hbm.at[pl.ds(0, CHUNK)], rows_v.at[b], gsems[b]).wait()
            pltpu.sync_copy(rows_v.at[b], acc.at[dst_v.at[j]], add=True)
            pltpu.async_copy(
                h_hbm.at[src_v.at[j + NBUF]], rows_v.at[b], gsems[b])
        return carry

    lax.fori_loop(0, CHUNKS // NBUF - 1, edge_body, 0)
    for b in range(NBUF):
        j = CHUNKS - NBUF + b
        pltpu.make_async_copy(
            h_hbm.at[pl.ds(0, CHUNK)], rows_v.at[b], gsems[b]).wait()
        pltpu.sync_copy(rows_v.at[b], acc.at[dst_v.at[j]], add=True)
    plsc.subcore_barrier()

    pltpu.sync_copy(acc.at[pl.ds(row0, ROWS_PER_TILE)],
                    out_hbm.at[c, pl.ds(row0, ROWS_PER_TILE)])


def kernel(x, edge_index, W1, W2, W3):
    src = edge_index[0].astype(jnp.int32)
    dst = edge_index[1].astype(jnp.int32)
    pad_idx = N + (jnp.arange(E_PAD - E, dtype=jnp.int32) % N_ZERO_ROWS)
    srcp = jnp.concatenate([src, pad_idx]).reshape(32, CHUNKS, CHUNK)
    dstp = jnp.concatenate([dst, pad_idx]).reshape(32, CHUNKS, CHUNK)
    h = _matmul(x, W1)
    p = _sc_aggregate(h, srcp, dstp)
    h = _matmul_relu(p, W2)
    p = _sc_aggregate(h, srcp, dstp)
    h = _matmul_relu(p, W3)
    p = _sc_aggregate(h, srcp, dstp)
    return _sum2(p)
